# baseline (speedup 1.0000x reference)
import jax
import jax.numpy as jnp
from jax import lax
from jax.experimental import pallas as pl
from jax.experimental.pallas import tpu as pltpu
from jax.experimental.pallas import tpu_sc as plsc

N = 10000
E = 160000
D = 256
NC = 2
NS = 16
NT = NC * NS
NPT = 320
TRASH = NPT
ACCR = NPT + 1
CAP = 8192
SCH = 2000
CH = 64
FCH = 16
BM = 1000

_mesh = plsc.VectorSubcoreMesh(core_axis_name="c", subcore_axis_name="s")
_params = pltpu.CompilerParams(needs_layout_passes=False)


def _mm_body(h_ref, wt_ref, b_ref, yl_ref, yr_ref):
    y = jnp.dot(h_ref[...], wt_ref[...], preferred_element_type=jnp.float32)
    yl_ref[...] = y[:, :D].astype(jnp.bfloat16)
    yr_ref[...] = y[:, D:] + b_ref[...]


def _mm(h, wt, b):
    return pl.pallas_call(
        _mm_body,
        grid=(N // BM,),
        in_specs=[
            pl.BlockSpec((BM, D), lambda i: (i, 0)),
            pl.BlockSpec((D, 2 * D), lambda i: (0, 0)),
            pl.BlockSpec((1, D), lambda i: (0, 0)),
        ],
        out_specs=[
            pl.BlockSpec((BM, D), lambda i: (i, 0)),
            pl.BlockSpec((BM, D), lambda i: (i, 0)),
        ],
        out_shape=[
            jax.ShapeDtypeStruct((N, D), jnp.bfloat16),
            jax.ShapeDtypeStruct((N, D), jnp.float32),
        ],
    )(h, wt, b)


def _bucket_body(src_hbm, dst_hbm, elist_hbm, ecnt_hbm, inv_hbm,
                 sbuf, dbuf, listbuf, hist, invb, cbuf):
    c = lax.axis_index("c")
    s = lax.axis_index("s")
    t = c * NS + s
    base = t * NPT
    lane = lax.iota(jnp.int32, 16)
    pad = jnp.full((16,), TRASH << 14, jnp.int32)

    def pre(i, _):
        listbuf[pl.ds(i * 16, 16)] = pad
        return 0
    lax.fori_loop(0, CAP // 16, pre, 0)

    def hz(i, _):
        hist[pl.ds(i * 16, 16)] = jnp.zeros((16,), jnp.float32)
        return 0
    lax.fori_loop(0, ACCR, hz, 0)

    def chunk(i, cursor):
        pltpu.sync_copy(src_hbm.at[pl.ds(i * SCH, SCH)], sbuf)
        pltpu.sync_copy(dst_hbm.at[pl.ds(i * SCH, SCH)], dbuf)

        def grp(q, cur):
            d = dbuf[pl.ds(q * 16, 16)]
            sv = sbuf[pl.ds(q * 16, 16)]
            loc = d - base
            m = (loc >= 0) & (loc < NPT)
            locs = jnp.where(m, loc, TRASH)
            packed = (locs << 14) | sv
            cm = plsc.cumsum(m.astype(jnp.int32))
            idx = jnp.minimum(jnp.full((16,), cur, jnp.int32) + cm - 1,
                              CAP - 1)
            plsc.store_scatter(listbuf, [idx], packed, mask=m)
            plsc.addupdate_scatter(hist, [locs * 16 + lane],
                                   m.astype(jnp.float32))
            return cur + cm[15]
        return lax.fori_loop(0, SCH // 16, grp, cursor)
    cursor = lax.fori_loop(0, E // SCH, chunk, jnp.int32(0))

    def inv_row(i, _):
        cnt = jnp.sum(hist[pl.ds(i * 16, 16)])
        invb[i, pl.ds(0, 16)] = jnp.full((16,), 1.0, jnp.float32) \
            / jnp.maximum(jnp.full((16,), cnt, jnp.float32), 1.0)
        return 0
    lax.fori_loop(0, NPT, inv_row, 0)

    pltpu.sync_copy(listbuf, elist_hbm.at[t])
    cbuf[pl.ds(0, 16)] = jnp.full((16,), jnp.minimum(cursor, CAP), jnp.int32)
    pltpu.sync_copy(cbuf, ecnt_hbm.at[t])
    pltpu.sync_copy(invb, inv_hbm.at[pl.ds(base, NPT)])


_bucket = pl.kernel(
    _bucket_body,
    out_type=(
        jax.ShapeDtypeStruct((NT, CAP), jnp.int32),
        jax.ShapeDtypeStruct((NT, 16), jnp.int32),
        jax.ShapeDtypeStruct((NT * NPT, 16), jnp.float32),
    ),
    mesh=_mesh,
    compiler_params=_params,
    scratch_types=[
        pltpu.VMEM((SCH,), jnp.int32),
        pltpu.VMEM((SCH,), jnp.int32),
        pltpu.VMEM((CAP,), jnp.int32),
        pltpu.VMEM((ACCR * 16,), jnp.float32),
        pltpu.VMEM((NPT, 16), jnp.float32),
        pltpu.VMEM((16,), jnp.int32),
    ],
)


def _make_layer(apply_relu):
    def body(yl_hbm, yr_hbm, elist_hbm, ecnt_hbm, inv_hbm, out_hbm,
             accflat, rowsA, rowsB, pkA, pkB, idxA, idxB, locA, locB, cbuf,
             ybuf, obuf, invbuf, semA, semB):
        c = lax.axis_index("c")
        s = lax.axis_index("s")
        t = c * NS + s
        base = t * NPT

        def zr(i, _):
            accflat[pl.ds(i * 16, 16)] = jnp.zeros((16,), jnp.float32)
            return 0
        lax.fori_loop(0, ACCR * (D // 16), zr, 0)

        pltpu.sync_copy(ecnt_hbm.at[t], cbuf)
        m = cbuf[pl.ds(0, 16)][0]
        nchunks = (m + CH - 1) // CH

        def start(k, pkb, idxb, locb, rb, sem):
            @pl.when(k < nchunks)
            def _():
                pltpu.sync_copy(elist_hbm.at[t, pl.ds(k * CH, CH)], pkb)
                for q in range(CH // 16):
                    pk = pkb[pl.ds(q * 16, 16)]
                    idxb[pl.ds(q * 16, 16)] = pk & 16383
                    locb[pl.ds(q * 16, 16)] = pk >> 14
                pltpu.make_async_copy(yl_hbm.at[idxb], rb, sem).start()

        def finish(k, idxb, locb, rb, sem):
            @pl.when(k < nchunks)
            def _():
                pltpu.make_async_copy(yl_hbm.at[idxb], rb, sem).wait()

                def qgrp(q, _):
                    addr = locb[pl.ds(q * 16, 16)] * D
                    a = [addr[ln] for ln in range(16)]
                    for ln in range(16):
                        e = q * 16 + ln
                        vals = [plsc.unpack(
                                    plsc.bitcast(rb[e, pl.ds(g2 * 16, 16)],
                                                 jnp.bfloat16),
                                    format=plsc.PackFormat.INTERLEAVED)
                                for g2 in range(D // 32)]
                        for g2 in range(D // 32):
                            ev, od = vals[g2]
                            plsc.addupdate(
                                accflat.at[pl.ds(a[ln] + g2 * 32, 16)], ev)
                            plsc.addupdate(
                                accflat.at[pl.ds(a[ln] + g2 * 32 + 16, 16)],
                                od)
                    return 0
                lax.fori_loop(0, CH // 16, qgrp, 0)

        start(jnp.int32(0), pkA, idxA, locA, rowsA, semA)
        start(jnp.int32(1), pkB, idxB, locB, rowsB, semB)

        def dpair(jj, _):
            j0 = 2 * jj
            finish(j0, idxA, locA, rowsA, semA)
            start(j0 + 2, pkA, idxA, locA, rowsA, semA)
            finish(j0 + 1, idxB, locB, rowsB, semB)
            start(j0 + 3, pkB, idxB, locB, rowsB, semB)
            return 0
        lax.fori_loop(0, (nchunks + 1) // 2, dpair, 0)

        nfc = jnp.minimum(NPT // FCH, (N - base + FCH - 1) // FCH)

        iota16 = lax.iota(jnp.int32, 16)
        evcols = [iota16 * 2 + g2 * 32 for g2 in range(D // 32)]
        odcols = [iota16 * 2 + (g2 * 32 + 1) for g2 in range(D // 32)]

        def fin(cb, _):
            n0 = base + cb * FCH
            l0 = cb * FCH
            pltpu.sync_copy(yr_hbm.at[pl.ds(n0, FCH)], ybuf)
            pltpu.sync_copy(inv_hbm.at[pl.ds(n0, FCH)], invbuf)

            def row(rr, _):
                iv = invbuf[rr, pl.ds(0, 16)]
                rows16 = jnp.full((16,), rr, jnp.int32)
                sums = [(accflat[pl.ds((l0 + rr) * D + g2 * 32, 16)],
                         accflat[pl.ds((l0 + rr) * D + g2 * 32 + 16, 16)])
                        for g2 in range(D // 32)]
                for g2 in range(D // 32):
                    ev, od = sums[g2]
                    plsc.store_scatter(obuf, [rows16, evcols[g2]], ev * iv)
                    plsc.store_scatter(obuf, [rows16, odcols[g2]], od * iv)
                return 0
            lax.fori_loop(0, FCH, row, 0)

            def row2(rr, _):
                for g in range(D // 16):
                    v = obuf[rr, pl.ds(g * 16, 16)] \
                        + ybuf[rr, pl.ds(g * 16, 16)]
                    if apply_relu:
                        v = jnp.maximum(v, 0.0)
                    obuf[rr, pl.ds(g * 16, 16)] = v
                return 0
            lax.fori_loop(0, FCH, row2, 0)
            pltpu.sync_copy(obuf, out_hbm.at[pl.ds(n0, FCH)])
            return 0
        lax.fori_loop(0, nfc, fin, 0)

    return pl.kernel(
        body,
        out_type=jax.ShapeDtypeStruct((N, D), jnp.float32),
        mesh=_mesh,
        compiler_params=_params,
        scratch_types=[
            pltpu.VMEM((ACCR * D,), jnp.float32),
            pltpu.VMEM((CH, D // 2), jnp.int32),
            pltpu.VMEM((CH, D // 2), jnp.int32),
            pltpu.VMEM((CH,), jnp.int32),
            pltpu.VMEM((CH,), jnp.int32),
            pltpu.VMEM((CH,), jnp.int32),
            pltpu.VMEM((CH,), jnp.int32),
            pltpu.VMEM((CH,), jnp.int32),
            pltpu.VMEM((CH,), jnp.int32),
            pltpu.VMEM((16,), jnp.int32),
            pltpu.VMEM((FCH, D), jnp.float32),
            pltpu.VMEM((FCH, D), jnp.float32),
            pltpu.VMEM((FCH, 16), jnp.float32),
            pltpu.SemaphoreType.DMA,
            pltpu.SemaphoreType.DMA,
        ],
    )


_layer_relu = _make_layer(True)
_layer_last = _make_layer(False)


def kernel(x, edge_index, W1l, b1l, W1r, W2l, b2l, W2r, W3l, b3l, W3r):
    ei = edge_index.astype(jnp.int32)
    src = ei[0]
    dst = ei[1]
    elist, ecnt, inv = _bucket(src, dst)
    h = x
    layers = [
        (W1l, b1l, W1r, _layer_relu),
        (W2l, b2l, W2r, _layer_relu),
        (W3l, b3l, W3r, _layer_last),
    ]
    for Wl, bl, Wr, layer in layers:
        wt = jnp.concatenate([Wl.T, Wr.T], axis=1)
        yl, yr = _mm(h, wt, bl.reshape(1, D))
        yl32 = lax.bitcast_convert_type(
            yl.reshape(N, D // 2, 2), jnp.int32)
        h = layer(yl32, yr, elist, ecnt, inv)
    return h


# speedup vs baseline: 1.2676x; 1.2676x over previous
import jax
import jax.numpy as jnp
from jax import lax
from jax.experimental import pallas as pl
from jax.experimental.pallas import tpu as pltpu
from jax.experimental.pallas import tpu_sc as plsc

N = 10000
E = 160000
D = 256
NC = 2
NS = 16
NT = NC * NS
NPT = 320
TRASH = NPT
ACCR = NPT + 1
CAP = 8192
SCH = 2000
CH = 64
FCH = 16
BM = 1000

_mesh = plsc.VectorSubcoreMesh(core_axis_name="c", subcore_axis_name="s")
_params = pltpu.CompilerParams(needs_layout_passes=False)


def _b16(v):
    bits = lax.bitcast_convert_type(v, jnp.int32)
    r = bits + 0x7FFF + ((bits >> 16) & 1)
    return (r >> 16) & 0xFFFF


def _mm_body(h_ref, wt_ref, b_ref, yl_ref, yr_ref):
    y = jnp.dot(h_ref[...], wt_ref[...], preferred_element_type=jnp.float32)
    yl_ref[...] = _b16(y[:, :D // 2]) | (_b16(y[:, D // 2:D]) << 16)
    yr_ref[...] = y[:, D:] + b_ref[...]


def _mm(h, wt, b):
    return pl.pallas_call(
        _mm_body,
        grid=(N // BM,),
        in_specs=[
            pl.BlockSpec((BM, D), lambda i: (i, 0)),
            pl.BlockSpec((D, 2 * D), lambda i: (0, 0)),
            pl.BlockSpec((1, D), lambda i: (0, 0)),
        ],
        out_specs=[
            pl.BlockSpec((BM, D // 2), lambda i: (i, 0)),
            pl.BlockSpec((BM, D), lambda i: (i, 0)),
        ],
        out_shape=[
            jax.ShapeDtypeStruct((N, D // 2), jnp.int32),
            jax.ShapeDtypeStruct((N, D), jnp.float32),
        ],
    )(h, wt, b)


def _bucket_body(src_hbm, dst_hbm, elist_hbm, ecnt_hbm, inv_hbm,
                 sbuf, dbuf, listbuf, hist, invb, cbuf):
    c = lax.axis_index("c")
    s = lax.axis_index("s")
    t = c * NS + s
    base = t * NPT
    lane = lax.iota(jnp.int32, 16)
    pad = jnp.full((16,), TRASH << 14, jnp.int32)

    def pre(i, _):
        listbuf[pl.ds(i * 16, 16)] = pad
        return 0
    lax.fori_loop(0, CAP // 16, pre, 0)

    def hz(i, _):
        hist[pl.ds(i * 16, 16)] = jnp.zeros((16,), jnp.float32)
        return 0
    lax.fori_loop(0, ACCR, hz, 0)

    def chunk(i, cursor):
        pltpu.sync_copy(src_hbm.at[pl.ds(i * SCH, SCH)], sbuf)
        pltpu.sync_copy(dst_hbm.at[pl.ds(i * SCH, SCH)], dbuf)

        def grp(q, cur):
            d = dbuf[pl.ds(q * 16, 16)]
            sv = sbuf[pl.ds(q * 16, 16)]
            loc = d - base
            m = (loc >= 0) & (loc < NPT)
            locs = jnp.where(m, loc, TRASH)
            packed = (locs << 14) | sv
            cm = plsc.cumsum(m.astype(jnp.int32))
            idx = jnp.minimum(jnp.full((16,), cur, jnp.int32) + cm - 1,
                              CAP - 1)
            plsc.store_scatter(listbuf, [idx], packed, mask=m)
            plsc.addupdate_scatter(hist, [locs * 16 + lane],
                                   m.astype(jnp.float32))
            return cur + cm[15]
        return lax.fori_loop(0, SCH // 16, grp, cursor)
    cursor = lax.fori_loop(0, E // SCH, chunk, jnp.int32(0))

    def inv_row(i, _):
        cnt = jnp.sum(hist[pl.ds(i * 16, 16)])
        invb[i, pl.ds(0, 16)] = jnp.full((16,), 1.0, jnp.float32) \
            / jnp.maximum(jnp.full((16,), cnt, jnp.float32), 1.0)
        return 0
    lax.fori_loop(0, NPT, inv_row, 0)

    pltpu.sync_copy(listbuf, elist_hbm.at[t])
    cbuf[pl.ds(0, 16)] = jnp.full((16,), jnp.minimum(cursor, CAP), jnp.int32)
    pltpu.sync_copy(cbuf, ecnt_hbm.at[t])
    pltpu.sync_copy(invb, inv_hbm.at[pl.ds(base, NPT)])


_bucket = pl.kernel(
    _bucket_body,
    out_type=(
        jax.ShapeDtypeStruct((NT, CAP), jnp.int32),
        jax.ShapeDtypeStruct((NT, 16), jnp.int32),
        jax.ShapeDtypeStruct((NT * NPT, 16), jnp.float32),
    ),
    mesh=_mesh,
    compiler_params=_params,
    scratch_types=[
        pltpu.VMEM((SCH,), jnp.int32),
        pltpu.VMEM((SCH,), jnp.int32),
        pltpu.VMEM((CAP,), jnp.int32),
        pltpu.VMEM((ACCR * 16,), jnp.float32),
        pltpu.VMEM((NPT, 16), jnp.float32),
        pltpu.VMEM((16,), jnp.int32),
    ],
)


def _make_layer(apply_relu):
    def body(yl_hbm, yr_hbm, elist_hbm, ecnt_hbm, inv_hbm, out_hbm,
             accflat, rowsA, rowsB, pkA, pkB, idxA, idxB, locA, locB, cbuf,
             ybuf, obuf, invbuf, semA, semB):
        c = lax.axis_index("c")
        s = lax.axis_index("s")
        t = c * NS + s
        base = t * NPT

        def zr(i, _):
            accflat[pl.ds(i * 16, 16)] = jnp.zeros((16,), jnp.float32)
            return 0
        lax.fori_loop(0, ACCR * (D // 16), zr, 0)

        pltpu.sync_copy(ecnt_hbm.at[t], cbuf)
        m = cbuf[pl.ds(0, 16)][0]
        nchunks = (m + CH - 1) // CH

        def start(k, pkb, idxb, locb, rb, sem):
            @pl.when(k < nchunks)
            def _():
                pltpu.sync_copy(elist_hbm.at[t, pl.ds(k * CH, CH)], pkb)
                for q in range(CH // 16):
                    pk = pkb[pl.ds(q * 16, 16)]
                    idxb[pl.ds(q * 16, 16)] = pk & 16383
                    locb[pl.ds(q * 16, 16)] = pk >> 14
                pltpu.make_async_copy(yl_hbm.at[idxb], rb, sem).start()

        def finish(k, idxb, locb, rb, sem):
            @pl.when(k < nchunks)
            def _():
                pltpu.make_async_copy(yl_hbm.at[idxb], rb, sem).wait()

                def qgrp(q, _):
                    addr = locb[pl.ds(q * 16, 16)] * D
                    a = [addr[ln] for ln in range(16)]
                    for ln in range(16):
                        e = q * 16 + ln
                        vals = [plsc.unpack(
                                    plsc.bitcast(rb[e, pl.ds(g2 * 16, 16)],
                                                 jnp.bfloat16),
                                    format=plsc.PackFormat.INTERLEAVED)
                                for g2 in range(D // 32)]
                        for g2 in range(D // 32):
                            lo, hi = vals[g2]
                            plsc.addupdate(
                                accflat.at[pl.ds(a[ln] + g2 * 16, 16)], lo)
                            plsc.addupdate(
                                accflat.at[pl.ds(a[ln] + D // 2 + g2 * 16,
                                                 16)], hi)
                    return 0
                lax.fori_loop(0, CH // 16, qgrp, 0)

        start(jnp.int32(0), pkA, idxA, locA, rowsA, semA)
        start(jnp.int32(1), pkB, idxB, locB, rowsB, semB)

        def dpair(jj, _):
            j0 = 2 * jj
            finish(j0, idxA, locA, rowsA, semA)
            start(j0 + 2, pkA, idxA, locA, rowsA, semA)
            finish(j0 + 1, idxB, locB, rowsB, semB)
            start(j0 + 3, pkB, idxB, locB, rowsB, semB)
            return 0
        lax.fori_loop(0, (nchunks + 1) // 2, dpair, 0)

        nfc = jnp.minimum(NPT // FCH, (N - base + FCH - 1) // FCH)

        def fin(cb, _):
            n0 = base + cb * FCH
            l0 = cb * FCH
            pltpu.sync_copy(yr_hbm.at[pl.ds(n0, FCH)], ybuf)
            pltpu.sync_copy(inv_hbm.at[pl.ds(n0, FCH)], invbuf)

            def row(rr, _):
                iv = invbuf[rr, pl.ds(0, 16)]
                sums = [accflat[pl.ds((l0 + rr) * D + g * 16, 16)]
                        for g in range(D // 16)]
                for g in range(D // 16):
                    v = sums[g] * iv + ybuf[rr, pl.ds(g * 16, 16)]
                    if apply_relu:
                        v = jnp.maximum(v, 0.0)
                    obuf[rr, pl.ds(g * 16, 16)] = v
                return 0
            lax.fori_loop(0, FCH, row, 0)
            pltpu.sync_copy(obuf, out_hbm.at[pl.ds(n0, FCH)])
            return 0
        lax.fori_loop(0, nfc, fin, 0)

    return pl.kernel(
        body,
        out_type=jax.ShapeDtypeStruct((N, D), jnp.float32),
        mesh=_mesh,
        compiler_params=_params,
        scratch_types=[
            pltpu.VMEM((ACCR * D,), jnp.float32),
            pltpu.VMEM((CH, D // 2), jnp.int32),
            pltpu.VMEM((CH, D // 2), jnp.int32),
            pltpu.VMEM((CH,), jnp.int32),
            pltpu.VMEM((CH,), jnp.int32),
            pltpu.VMEM((CH,), jnp.int32),
            pltpu.VMEM((CH,), jnp.int32),
            pltpu.VMEM((CH,), jnp.int32),
            pltpu.VMEM((CH,), jnp.int32),
            pltpu.VMEM((16,), jnp.int32),
            pltpu.VMEM((FCH, D), jnp.float32),
            pltpu.VMEM((FCH, D), jnp.float32),
            pltpu.VMEM((FCH, 16), jnp.float32),
            pltpu.SemaphoreType.DMA,
            pltpu.SemaphoreType.DMA,
        ],
    )


_layer_relu = _make_layer(True)
_layer_last = _make_layer(False)


def kernel(x, edge_index, W1l, b1l, W1r, W2l, b2l, W2r, W3l, b3l, W3r):
    ei = edge_index.astype(jnp.int32)
    src = ei[0]
    dst = ei[1]
    elist, ecnt, inv = _bucket(src, dst)
    h = x
    layers = [
        (W1l, b1l, W1r, _layer_relu),
        (W2l, b2l, W2r, _layer_relu),
        (W3l, b3l, W3r, _layer_last),
    ]
    for Wl, bl, Wr, layer in layers:
        wt = jnp.concatenate([Wl.T, Wr.T], axis=1)
        yl, yr = _mm(h, wt, bl.reshape(1, D))
        h = layer(yl, yr, elist, ecnt, inv)
    return h


# speedup vs baseline: 1.2754x; 1.0061x over previous
import jax
import jax.numpy as jnp
from jax import lax
from jax.experimental import pallas as pl
from jax.experimental.pallas import tpu as pltpu
from jax.experimental.pallas import tpu_sc as plsc

N = 10000
E = 160000
D = 256
NC = 2
NS = 16
NT = NC * NS
NPT = 320
TRASH = NPT
ACCR = NPT + 1
CAP = 8192
SCH = 2000
CH = 64
FCH = 16
BM = 1000

_mesh = plsc.VectorSubcoreMesh(core_axis_name="c", subcore_axis_name="s")
_params = pltpu.CompilerParams(needs_layout_passes=False)


def _b16(v):
    bits = lax.bitcast_convert_type(v, jnp.int32)
    r = bits + 0x7FFF + ((bits >> 16) & 1)
    return (r >> 16) & 0xFFFF


def _mm_body(h_ref, wt_ref, b_ref, yl_ref, yr_ref):
    y = jnp.dot(h_ref[...], wt_ref[...], preferred_element_type=jnp.float32)
    yl_ref[...] = _b16(y[:, :D // 2]) | (_b16(y[:, D // 2:D]) << 16)
    yr_ref[...] = y[:, D:] + b_ref[...]


def _mm(h, wt, b):
    return pl.pallas_call(
        _mm_body,
        grid=(N // BM,),
        in_specs=[
            pl.BlockSpec((BM, D), lambda i: (i, 0)),
            pl.BlockSpec((D, 2 * D), lambda i: (0, 0)),
            pl.BlockSpec((1, D), lambda i: (0, 0)),
        ],
        out_specs=[
            pl.BlockSpec((BM, D // 2), lambda i: (i, 0)),
            pl.BlockSpec((BM, D), lambda i: (i, 0)),
        ],
        out_shape=[
            jax.ShapeDtypeStruct((N, D // 2), jnp.int32),
            jax.ShapeDtypeStruct((N, D), jnp.float32),
        ],
    )(h, wt, b)


def _bucket_body(src_hbm, dst_hbm, elist_hbm, ecnt_hbm, inv_hbm,
                 sbuf, dbuf, listbuf, hist, invb, cbuf, pbuf, cmbuf, tsbuf,
                 mbuf):
    c = lax.axis_index("c")
    s = lax.axis_index("s")
    t = c * NS + s
    base = t * NPT
    lane = lax.iota(jnp.int32, 16)
    pad = jnp.full((16,), TRASH << 14, jnp.int32)

    def pre(i, _):
        listbuf[pl.ds(i * 16, 16)] = pad
        return 0
    lax.fori_loop(0, CAP // 16, pre, 0)

    def hz(i, _):
        hist[pl.ds(i * 16, 16)] = jnp.zeros((16,), jnp.float32)
        return 0
    lax.fori_loop(0, ACCR, hz, 0)

    def chunk(i, running):
        pltpu.sync_copy(src_hbm.at[pl.ds(i * SCH, SCH)], sbuf)
        pltpu.sync_copy(dst_hbm.at[pl.ds(i * SCH, SCH)], dbuf)

        def p1(q, _):
            d = dbuf[pl.ds(q * 16, 16)]
            sv = sbuf[pl.ds(q * 16, 16)]
            loc = d - base
            m = (loc >= 0) & (loc < NPT)
            locs = jnp.where(m, loc, TRASH)
            cm = plsc.cumsum(m.astype(jnp.int32))
            pbuf[pl.ds(q * 16, 16)] = (locs << 14) | sv
            mbuf[pl.ds(q * 16, 16)] = m.astype(jnp.int32)
            cmbuf[pl.ds(q * 16, 16)] = cm
            tsbuf[pl.ds(q * 16, 16)] = plsc.cummax(lax.rev(cm, (0,)))
            return 0
        lax.fori_loop(0, SCH // 16, p1, 0)

        def p2(q, run):
            cm = cmbuf[pl.ds(q * 16, 16)]
            packed = pbuf[pl.ds(q * 16, 16)]
            mi = mbuf[pl.ds(q * 16, 16)]
            ts = tsbuf[pl.ds(q * 16, 16)]
            idx = jnp.minimum(run + cm - 1, CAP - 1)
            plsc.store_scatter(listbuf, [idx], packed, mask=mi > 0)
            return run + ts
        return lax.fori_loop(0, SCH // 16, p2, running)
    running = lax.fori_loop(0, E // SCH, chunk, jnp.zeros((16,), jnp.int32))
    cursor = running[0]

    def hsc(i, _):
        loc = listbuf[pl.ds(i * 16, 16)] >> 14
        plsc.addupdate_scatter(hist, [loc * 16 + lane],
                               jnp.ones((16,), jnp.float32))
        return 0
    lax.fori_loop(0, CAP // 16, hsc, 0)

    def inv_row(i, _):
        cnt = jnp.sum(hist[pl.ds(i * 16, 16)])
        invb[i, pl.ds(0, 16)] = jnp.full((16,), 1.0, jnp.float32) \
            / jnp.maximum(jnp.full((16,), cnt, jnp.float32), 1.0)
        return 0
    lax.fori_loop(0, NPT, inv_row, 0)

    pltpu.sync_copy(listbuf, elist_hbm.at[t])
    cbuf[pl.ds(0, 16)] = jnp.full((16,), jnp.minimum(cursor, CAP), jnp.int32)
    pltpu.sync_copy(cbuf, ecnt_hbm.at[t])
    pltpu.sync_copy(invb, inv_hbm.at[pl.ds(base, NPT)])


_bucket = pl.kernel(
    _bucket_body,
    out_type=(
        jax.ShapeDtypeStruct((NT, CAP), jnp.int32),
        jax.ShapeDtypeStruct((NT, 16), jnp.int32),
        jax.ShapeDtypeStruct((NT * NPT, 16), jnp.float32),
    ),
    mesh=_mesh,
    compiler_params=_params,
    scratch_types=[
        pltpu.VMEM((SCH,), jnp.int32),
        pltpu.VMEM((SCH,), jnp.int32),
        pltpu.VMEM((CAP,), jnp.int32),
        pltpu.VMEM((ACCR * 16,), jnp.float32),
        pltpu.VMEM((NPT, 16), jnp.float32),
        pltpu.VMEM((16,), jnp.int32),
        pltpu.VMEM((SCH,), jnp.int32),
        pltpu.VMEM((SCH,), jnp.int32),
        pltpu.VMEM((SCH,), jnp.int32),
        pltpu.VMEM((SCH,), jnp.int32),
    ],
)


def _make_layer(apply_relu):
    def body(yl_hbm, yr_hbm, elist_hbm, ecnt_hbm, inv_hbm, out_hbm,
             accflat, rowsA, rowsB, pkA, pkB, idxA, idxB, locA, locB, cbuf,
             ybuf, obuf, invbuf, semA, semB):
        c = lax.axis_index("c")
        s = lax.axis_index("s")
        t = c * NS + s
        base = t * NPT

        def zr(i, _):
            accflat[pl.ds(i * 16, 16)] = jnp.zeros((16,), jnp.float32)
            return 0
        lax.fori_loop(0, ACCR * (D // 16), zr, 0)

        pltpu.sync_copy(ecnt_hbm.at[t], cbuf)
        m = cbuf[pl.ds(0, 16)][0]
        nchunks = (m + CH - 1) // CH

        def start(k, pkb, idxb, locb, rb, sem):
            @pl.when(k < nchunks)
            def _():
                pltpu.sync_copy(elist_hbm.at[t, pl.ds(k * CH, CH)], pkb)
                for q in range(CH // 16):
                    pk = pkb[pl.ds(q * 16, 16)]
                    idxb[pl.ds(q * 16, 16)] = pk & 16383
                    locb[pl.ds(q * 16, 16)] = pk >> 14
                pltpu.make_async_copy(yl_hbm.at[idxb], rb, sem).start()

        def finish(k, idxb, locb, rb, sem):
            @pl.when(k < nchunks)
            def _():
                pltpu.make_async_copy(yl_hbm.at[idxb], rb, sem).wait()

                def qgrp(q, _):
                    addr = locb[pl.ds(q * 16, 16)] * D
                    a = [addr[ln] for ln in range(16)]
                    for ln in range(16):
                        e = q * 16 + ln
                        vals = [plsc.unpack(
                                    plsc.bitcast(rb[e, pl.ds(g2 * 16, 16)],
                                                 jnp.bfloat16),
                                    format=plsc.PackFormat.INTERLEAVED)
                                for g2 in range(D // 32)]
                        for g2 in range(D // 32):
                            lo, hi = vals[g2]
                            plsc.addupdate(
                                accflat.at[pl.ds(a[ln] + g2 * 16, 16)], lo)
                            plsc.addupdate(
                                accflat.at[pl.ds(a[ln] + D // 2 + g2 * 16,
                                                 16)], hi)
                    return 0
                lax.fori_loop(0, CH // 16, qgrp, 0)

        start(jnp.int32(0), pkA, idxA, locA, rowsA, semA)
        start(jnp.int32(1), pkB, idxB, locB, rowsB, semB)

        def dpair(jj, _):
            j0 = 2 * jj
            finish(j0, idxA, locA, rowsA, semA)
            start(j0 + 2, pkA, idxA, locA, rowsA, semA)
            finish(j0 + 1, idxB, locB, rowsB, semB)
            start(j0 + 3, pkB, idxB, locB, rowsB, semB)
            return 0
        lax.fori_loop(0, (nchunks + 1) // 2, dpair, 0)

        nfc = jnp.minimum(NPT // FCH, (N - base + FCH - 1) // FCH)

        def fin(cb, _):
            n0 = base + cb * FCH
            l0 = cb * FCH
            pltpu.sync_copy(yr_hbm.at[pl.ds(n0, FCH)], ybuf)
            pltpu.sync_copy(inv_hbm.at[pl.ds(n0, FCH)], invbuf)

            def row(rr, _):
                iv = invbuf[rr, pl.ds(0, 16)]
                sums = [accflat[pl.ds((l0 + rr) * D + g * 16, 16)]
                        for g in range(D // 16)]
                for g in range(D // 16):
                    v = sums[g] * iv + ybuf[rr, pl.ds(g * 16, 16)]
                    if apply_relu:
                        v = jnp.maximum(v, 0.0)
                    obuf[rr, pl.ds(g * 16, 16)] = v
                return 0
            lax.fori_loop(0, FCH, row, 0)
            pltpu.sync_copy(obuf, out_hbm.at[pl.ds(n0, FCH)])
            return 0
        lax.fori_loop(0, nfc, fin, 0)

    return pl.kernel(
        body,
        out_type=jax.ShapeDtypeStruct((N, D), jnp.float32),
        mesh=_mesh,
        compiler_params=_params,
        scratch_types=[
            pltpu.VMEM((ACCR * D,), jnp.float32),
            pltpu.VMEM((CH, D // 2), jnp.int32),
            pltpu.VMEM((CH, D // 2), jnp.int32),
            pltpu.VMEM((CH,), jnp.int32),
            pltpu.VMEM((CH,), jnp.int32),
            pltpu.VMEM((CH,), jnp.int32),
            pltpu.VMEM((CH,), jnp.int32),
            pltpu.VMEM((CH,), jnp.int32),
            pltpu.VMEM((CH,), jnp.int32),
            pltpu.VMEM((16,), jnp.int32),
            pltpu.VMEM((FCH, D), jnp.float32),
            pltpu.VMEM((FCH, D), jnp.float32),
            pltpu.VMEM((FCH, 16), jnp.float32),
            pltpu.SemaphoreType.DMA,
            pltpu.SemaphoreType.DMA,
        ],
    )


_layer_relu = _make_layer(True)
_layer_last = _make_layer(False)


def kernel(x, edge_index, W1l, b1l, W1r, W2l, b2l, W2r, W3l, b3l, W3r):
    ei = edge_index.astype(jnp.int32)
    src = ei[0]
    dst = ei[1]
    elist, ecnt, inv = _bucket(src, dst)
    h = x
    layers = [
        (W1l, b1l, W1r, _layer_relu),
        (W2l, b2l, W2r, _layer_relu),
        (W3l, b3l, W3r, _layer_last),
    ]
    for Wl, bl, Wr, layer in layers:
        wt = jnp.concatenate([Wl.T, Wr.T], axis=1)
        yl, yr = _mm(h, wt, bl.reshape(1, D))
        h = layer(yl, yr, elist, ecnt, inv)
    return h


# speedup vs baseline: 1.5211x; 1.1927x over previous
import jax
import jax.numpy as jnp
from jax import lax
from jax.experimental import pallas as pl
from jax.experimental.pallas import tpu as pltpu
from jax.experimental.pallas import tpu_sc as plsc

N = 10000
E = 160000
D = 256
NC = 2
NS = 16
NT = NC * NS
NPT = 320
TRASH = NPT
ACCR = NPT + 1
CAP = 8192
SCH = 8000
CH = 64
FCH = 16
BM = 1000

_mesh = plsc.VectorSubcoreMesh(core_axis_name="c", subcore_axis_name="s")
_params = pltpu.CompilerParams(needs_layout_passes=False)


def _b16(v):
    bits = lax.bitcast_convert_type(v, jnp.int32)
    r = bits + 0x7FFF + ((bits >> 16) & 1)
    return (r >> 16) & 0xFFFF


def _mm_body(h_ref, wt_ref, b_ref, yl_ref, yr_ref):
    y = jnp.dot(h_ref[...], wt_ref[...], preferred_element_type=jnp.float32)
    yl_ref[...] = _b16(y[:, :D // 2]) | (_b16(y[:, D // 2:D]) << 16)
    yr_ref[...] = y[:, D:] + b_ref[...]


def _mm(h, wt, b):
    return pl.pallas_call(
        _mm_body,
        grid=(N // BM,),
        in_specs=[
            pl.BlockSpec((BM, D), lambda i: (i, 0)),
            pl.BlockSpec((D, 2 * D), lambda i: (0, 0)),
            pl.BlockSpec((1, D), lambda i: (0, 0)),
        ],
        out_specs=[
            pl.BlockSpec((BM, D // 2), lambda i: (i, 0)),
            pl.BlockSpec((BM, D), lambda i: (i, 0)),
        ],
        out_shape=[
            jax.ShapeDtypeStruct((N, D // 2), jnp.int32),
            jax.ShapeDtypeStruct((N, D), jnp.float32),
        ],
    )(h, wt, b)


def _bucket_body(src_hbm, dst_hbm, elist_hbm, ecnt_hbm, inv_hbm,
                 sbuf, dbuf, listbuf, hist, invb, cbuf, pbuf, cmbuf, tsbuf,
                 mbuf):
    c = lax.axis_index("c")
    s = lax.axis_index("s")
    t = c * NS + s
    base = t * NPT
    lane = lax.iota(jnp.int32, 16)
    pad = jnp.full((16,), TRASH << 14, jnp.int32)

    def pre(i, _):
        listbuf[pl.ds(i * 16, 16)] = pad
        return 0
    lax.fori_loop(0, CAP // 16, pre, 0)

    def hz(i, _):
        hist[pl.ds(i * 16, 16)] = jnp.zeros((16,), jnp.float32)
        return 0
    lax.fori_loop(0, ACCR, hz, 0)

    def chunk(i, running):
        pltpu.sync_copy(src_hbm.at[pl.ds(i * SCH, SCH)], sbuf)
        pltpu.sync_copy(dst_hbm.at[pl.ds(i * SCH, SCH)], dbuf)

        def p1(q, _):
            d = dbuf[pl.ds(q * 16, 16)]
            sv = sbuf[pl.ds(q * 16, 16)]
            loc = d - base
            m = (loc >= 0) & (loc < NPT)
            locs = jnp.where(m, loc, TRASH)
            cm = plsc.cumsum(m.astype(jnp.int32))
            pbuf[pl.ds(q * 16, 16)] = (locs << 14) | sv
            mbuf[pl.ds(q * 16, 16)] = m.astype(jnp.int32)
            cmbuf[pl.ds(q * 16, 16)] = cm
            tsbuf[pl.ds(q * 16, 16)] = plsc.cummax(lax.rev(cm, (0,)))
            return 0
        lax.fori_loop(0, SCH // 16, p1, 0)

        def p2(q, run):
            cm = cmbuf[pl.ds(q * 16, 16)]
            packed = pbuf[pl.ds(q * 16, 16)]
            mi = mbuf[pl.ds(q * 16, 16)]
            ts = tsbuf[pl.ds(q * 16, 16)]
            idx = jnp.minimum(run + cm - 1, CAP - 1)
            plsc.store_scatter(listbuf, [idx], packed, mask=mi > 0)
            return run + ts
        return lax.fori_loop(0, SCH // 16, p2, running)
    running = lax.fori_loop(0, E // SCH, chunk, jnp.zeros((16,), jnp.int32))
    cursor = running[0]

    def hsc(i, _):
        loc = listbuf[pl.ds(i * 16, 16)] >> 14
        plsc.addupdate_scatter(hist, [loc * 16 + lane],
                               jnp.ones((16,), jnp.float32))
        return 0
    lax.fori_loop(0, CAP // 16, hsc, 0)

    def inv_row(i, _):
        cnt = jnp.sum(hist[pl.ds(i * 16, 16)])
        invb[i, pl.ds(0, 16)] = jnp.full((16,), 1.0, jnp.float32) \
            / jnp.maximum(jnp.full((16,), cnt, jnp.float32), 1.0)
        return 0
    lax.fori_loop(0, NPT, inv_row, 0)

    pltpu.sync_copy(listbuf, elist_hbm.at[t])
    cbuf[pl.ds(0, 16)] = jnp.full((16,), jnp.minimum(cursor, CAP), jnp.int32)
    pltpu.sync_copy(cbuf, ecnt_hbm.at[t])
    pltpu.sync_copy(invb, inv_hbm.at[pl.ds(base, NPT)])


_bucket = pl.kernel(
    _bucket_body,
    out_type=(
        jax.ShapeDtypeStruct((NT, CAP), jnp.int32),
        jax.ShapeDtypeStruct((NT, 16), jnp.int32),
        jax.ShapeDtypeStruct((NT * NPT, 16), jnp.float32),
    ),
    mesh=_mesh,
    compiler_params=_params,
    scratch_types=[
        pltpu.VMEM((SCH,), jnp.int32),
        pltpu.VMEM((SCH,), jnp.int32),
        pltpu.VMEM((CAP,), jnp.int32),
        pltpu.VMEM((ACCR * 16,), jnp.float32),
        pltpu.VMEM((NPT, 16), jnp.float32),
        pltpu.VMEM((16,), jnp.int32),
        pltpu.VMEM((SCH,), jnp.int32),
        pltpu.VMEM((SCH,), jnp.int32),
        pltpu.VMEM((SCH,), jnp.int32),
        pltpu.VMEM((SCH,), jnp.int32),
    ],
)


def _make_layer(apply_relu):
    def body(yl_hbm, yr_hbm, elist_hbm, ecnt_hbm, inv_hbm, out_hbm,
             accflat, rowsA, rowsB, lbuf, idxA, idxB, locA, locB, cbuf,
             ybuf, obuf, invbuf, semA, semB):
        c = lax.axis_index("c")
        s = lax.axis_index("s")
        t = c * NS + s
        base = t * NPT

        def zr(i, _):
            accflat[pl.ds(i * 16, 16)] = jnp.zeros((16,), jnp.float32)
            return 0
        lax.fori_loop(0, ACCR * (D // 16), zr, 0)

        pltpu.sync_copy(ecnt_hbm.at[t], cbuf)
        m = cbuf[pl.ds(0, 16)][0]
        nchunks = (m + CH - 1) // CH
        pltpu.sync_copy(elist_hbm.at[t], lbuf)

        def start(k, pkb, idxb, locb, rb, sem):
            @pl.when(k < nchunks)
            def _():
                for q in range(CH // 16):
                    pk = lbuf[pl.ds(k * CH + q * 16, 16)]
                    idxb[pl.ds(q * 16, 16)] = pk & 16383
                    locb[pl.ds(q * 16, 16)] = pk >> 14
                pltpu.make_async_copy(yl_hbm.at[idxb], rb, sem).start()

        def finish(k, idxb, locb, rb, sem):
            @pl.when(k < nchunks)
            def _():
                pltpu.make_async_copy(yl_hbm.at[idxb], rb, sem).wait()

                def qgrp(q, _):
                    addr = locb[pl.ds(q * 16, 16)] * D
                    a = [addr[ln] for ln in range(16)]
                    for ln in range(16):
                        e = q * 16 + ln
                        vals = [plsc.unpack(
                                    plsc.bitcast(rb[e, pl.ds(g2 * 16, 16)],
                                                 jnp.bfloat16),
                                    format=plsc.PackFormat.INTERLEAVED)
                                for g2 in range(D // 32)]
                        for g2 in range(D // 32):
                            lo, hi = vals[g2]
                            plsc.addupdate(
                                accflat.at[pl.ds(a[ln] + g2 * 16, 16)], lo)
                            plsc.addupdate(
                                accflat.at[pl.ds(a[ln] + D // 2 + g2 * 16,
                                                 16)], hi)
                    return 0
                lax.fori_loop(0, CH // 16, qgrp, 0)

        start(jnp.int32(0), None, idxA, locA, rowsA, semA)
        start(jnp.int32(1), None, idxB, locB, rowsB, semB)

        def dpair(jj, _):
            j0 = 2 * jj
            finish(j0, idxA, locA, rowsA, semA)
            start(j0 + 2, None, idxA, locA, rowsA, semA)
            finish(j0 + 1, idxB, locB, rowsB, semB)
            start(j0 + 3, None, idxB, locB, rowsB, semB)
            return 0
        lax.fori_loop(0, (nchunks + 1) // 2, dpair, 0)

        nfc = jnp.minimum(NPT // FCH, (N - base + FCH - 1) // FCH)

        def fin(cb, _):
            n0 = base + cb * FCH
            l0 = cb * FCH
            pltpu.sync_copy(yr_hbm.at[pl.ds(n0, FCH)], ybuf)
            pltpu.sync_copy(inv_hbm.at[pl.ds(n0, FCH)], invbuf)

            def row(rr, _):
                iv = invbuf[rr, pl.ds(0, 16)]
                sums = [accflat[pl.ds((l0 + rr) * D + g * 16, 16)]
                        for g in range(D // 16)]
                for g in range(D // 16):
                    v = sums[g] * iv + ybuf[rr, pl.ds(g * 16, 16)]
                    if apply_relu:
                        v = jnp.maximum(v, 0.0)
                    obuf[rr, pl.ds(g * 16, 16)] = v
                return 0
            lax.fori_loop(0, FCH, row, 0)
            pltpu.sync_copy(obuf, out_hbm.at[pl.ds(n0, FCH)])
            return 0
        lax.fori_loop(0, nfc, fin, 0)

    return pl.kernel(
        body,
        out_type=jax.ShapeDtypeStruct((N, D), jnp.float32),
        mesh=_mesh,
        compiler_params=_params,
        scratch_types=[
            pltpu.VMEM((ACCR * D,), jnp.float32),
            pltpu.VMEM((CH, D // 2), jnp.int32),
            pltpu.VMEM((CH, D // 2), jnp.int32),
            pltpu.VMEM((CAP,), jnp.int32),
            pltpu.VMEM((CH,), jnp.int32),
            pltpu.VMEM((CH,), jnp.int32),
            pltpu.VMEM((CH,), jnp.int32),
            pltpu.VMEM((CH,), jnp.int32),
            pltpu.VMEM((16,), jnp.int32),
            pltpu.VMEM((FCH, D), jnp.float32),
            pltpu.VMEM((FCH, D), jnp.float32),
            pltpu.VMEM((FCH, 16), jnp.float32),
            pltpu.SemaphoreType.DMA,
            pltpu.SemaphoreType.DMA,
        ],
    )


_layer_relu = _make_layer(True)
_layer_last = _make_layer(False)


def kernel(x, edge_index, W1l, b1l, W1r, W2l, b2l, W2r, W3l, b3l, W3r):
    ei = edge_index.astype(jnp.int32)
    src = ei[0]
    dst = ei[1]
    elist, ecnt, inv = _bucket(src, dst)
    h = x
    layers = [
        (W1l, b1l, W1r, _layer_relu),
        (W2l, b2l, W2r, _layer_relu),
        (W3l, b3l, W3r, _layer_last),
    ]
    for Wl, bl, Wr, layer in layers:
        wt = jnp.concatenate([Wl.T, Wr.T], axis=1)
        yl, yr = _mm(h, wt, bl.reshape(1, D))
        h = layer(yl, yr, elist, ecnt, inv)
    return h


# speedup vs baseline: 1.6712x; 1.0987x over previous
import jax
import jax.numpy as jnp
from jax import lax
from jax.experimental import pallas as pl
from jax.experimental.pallas import tpu as pltpu
from jax.experimental.pallas import tpu_sc as plsc

N = 10000
E = 160000
D = 256
NC = 2
NS = 16
NT = NC * NS
NPT = 320
TRASH = NPT
ACCR = NPT + 1
CAP = 8192
SCH = 8000
CH = 64
FCH = 16
BM = 1000

_mesh = plsc.VectorSubcoreMesh(core_axis_name="c", subcore_axis_name="s")
_params = pltpu.CompilerParams(needs_layout_passes=False)


def _b16(v):
    bits = lax.bitcast_convert_type(v, jnp.int32)
    r = bits + 0x7FFF + ((bits >> 16) & 1)
    return (r >> 16) & 0xFFFF


def _mm_body(h_ref, wt_ref, b_ref, yl_ref, yr_ref):
    y = jnp.dot(h_ref[...], wt_ref[...], preferred_element_type=jnp.float32)
    yl_ref[...] = _b16(y[:, :D // 2]) | (_b16(y[:, D // 2:D]) << 16)
    yr_ref[...] = y[:, D:] + b_ref[...]


def _mm(h, wt, b):
    return pl.pallas_call(
        _mm_body,
        grid=(N // BM,),
        in_specs=[
            pl.BlockSpec((BM, D), lambda i: (i, 0)),
            pl.BlockSpec((D, 2 * D), lambda i: (0, 0)),
            pl.BlockSpec((1, D), lambda i: (0, 0)),
        ],
        out_specs=[
            pl.BlockSpec((BM, D // 2), lambda i: (i, 0)),
            pl.BlockSpec((BM, D), lambda i: (i, 0)),
        ],
        out_shape=[
            jax.ShapeDtypeStruct((N, D // 2), jnp.int32),
            jax.ShapeDtypeStruct((N, D), jnp.float32),
        ],
    )(h, wt, b)


def _bucket_body(src_hbm, dst_hbm, elist_hbm, ecnt_hbm, inv_hbm,
                 sbuf, dbuf, listbuf, hist, invb, cbuf, pbuf, cmbuf, tsbuf,
                 mbuf):
    c = lax.axis_index("c")
    s = lax.axis_index("s")
    t = c * NS + s
    base = t * NPT
    lane = lax.iota(jnp.int32, 16)
    pad = jnp.full((16,), TRASH << 14, jnp.int32)

    def pre(i, _):
        listbuf[pl.ds(i * 16, 16)] = pad
        return 0
    lax.fori_loop(0, CAP // 16, pre, 0)

    def hz(i, _):
        hist[pl.ds(i * 16, 16)] = jnp.zeros((16,), jnp.float32)
        return 0
    lax.fori_loop(0, ACCR, hz, 0)

    def chunk(i, running):
        pltpu.sync_copy(src_hbm.at[pl.ds(i * SCH, SCH)], sbuf)
        pltpu.sync_copy(dst_hbm.at[pl.ds(i * SCH, SCH)], dbuf)

        def p1(qq, _):
            qs = [qq * 4, qq * 4 + 1, qq * 4 + 2, qq * 4 + 3]
            ds_ = [dbuf[pl.ds(q * 16, 16)] for q in qs]
            ss = [sbuf[pl.ds(q * 16, 16)] for q in qs]
            for j, q in enumerate(qs):
                loc = ds_[j] - base
                m = (loc >= 0) & (loc < NPT)
                locs = jnp.where(m, loc, TRASH)
                cm = plsc.cumsum(m.astype(jnp.int32))
                pbuf[pl.ds(q * 16, 16)] = (locs << 14) | ss[j]
                mbuf[pl.ds(q * 16, 16)] = m.astype(jnp.int32)
                cmbuf[pl.ds(q * 16, 16)] = cm
                tsbuf[pl.ds(q * 16, 16)] = plsc.cummax(lax.rev(cm, (0,)))
            return 0
        lax.fori_loop(0, SCH // 64, p1, 0)

        def p2(qq, run):
            qs = [qq * 4, qq * 4 + 1, qq * 4 + 2, qq * 4 + 3]
            cms = [cmbuf[pl.ds(q * 16, 16)] for q in qs]
            pks = [pbuf[pl.ds(q * 16, 16)] for q in qs]
            mis = [mbuf[pl.ds(q * 16, 16)] for q in qs]
            tss = [tsbuf[pl.ds(q * 16, 16)] for q in qs]
            for j in range(4):
                idx = jnp.minimum(run + cms[j] - 1, CAP - 1)
                plsc.store_scatter(listbuf, [idx], pks[j], mask=mis[j] > 0)
                run = run + tss[j]
            return run
        return lax.fori_loop(0, SCH // 64, p2, running)
    running = lax.fori_loop(0, E // SCH, chunk, jnp.zeros((16,), jnp.int32))
    cursor = running[0]

    def hsc(i, _):
        loc = listbuf[pl.ds(i * 16, 16)] >> 14
        plsc.addupdate_scatter(hist, [loc * 16 + lane],
                               jnp.ones((16,), jnp.float32))
        return 0
    lax.fori_loop(0, CAP // 16, hsc, 0)

    def inv_row(i, _):
        cnt = jnp.sum(hist[pl.ds(i * 16, 16)])
        invb[i, pl.ds(0, 16)] = jnp.full((16,), 1.0, jnp.float32) \
            / jnp.maximum(jnp.full((16,), cnt, jnp.float32), 1.0)
        return 0
    lax.fori_loop(0, NPT, inv_row, 0)

    pltpu.sync_copy(listbuf, elist_hbm.at[t])
    cbuf[pl.ds(0, 16)] = jnp.full((16,), jnp.minimum(cursor, CAP), jnp.int32)
    pltpu.sync_copy(cbuf, ecnt_hbm.at[t])
    pltpu.sync_copy(invb, inv_hbm.at[pl.ds(base, NPT)])


_bucket = pl.kernel(
    _bucket_body,
    out_type=(
        jax.ShapeDtypeStruct((NT, CAP), jnp.int32),
        jax.ShapeDtypeStruct((NT, 16), jnp.int32),
        jax.ShapeDtypeStruct((NT * NPT, 16), jnp.float32),
    ),
    mesh=_mesh,
    compiler_params=_params,
    scratch_types=[
        pltpu.VMEM((SCH,), jnp.int32),
        pltpu.VMEM((SCH,), jnp.int32),
        pltpu.VMEM((CAP,), jnp.int32),
        pltpu.VMEM((ACCR * 16,), jnp.float32),
        pltpu.VMEM((NPT, 16), jnp.float32),
        pltpu.VMEM((16,), jnp.int32),
        pltpu.VMEM((SCH,), jnp.int32),
        pltpu.VMEM((SCH,), jnp.int32),
        pltpu.VMEM((SCH,), jnp.int32),
        pltpu.VMEM((SCH,), jnp.int32),
    ],
)


def _make_layer(apply_relu):
    def body(yl_hbm, yr_hbm, elist_hbm, ecnt_hbm, inv_hbm, out_hbm,
             accflat, rowsA, rowsB, lbuf, idxA, idxB, locA, locB, cbuf,
             ybuf, obuf, invbuf, semA, semB):
        c = lax.axis_index("c")
        s = lax.axis_index("s")
        t = c * NS + s
        base = t * NPT

        def zr(i, _):
            accflat[pl.ds(i * 16, 16)] = jnp.zeros((16,), jnp.float32)
            return 0
        lax.fori_loop(0, ACCR * (D // 16), zr, 0)

        pltpu.sync_copy(ecnt_hbm.at[t], cbuf)
        m = cbuf[pl.ds(0, 16)][0]
        nchunks = (m + CH - 1) // CH
        pltpu.sync_copy(elist_hbm.at[t], lbuf)

        def start(k, pkb, idxb, locb, rb, sem):
            @pl.when(k < nchunks)
            def _():
                for q in range(CH // 16):
                    pk = lbuf[pl.ds(k * CH + q * 16, 16)]
                    idxb[pl.ds(q * 16, 16)] = pk & 16383
                    locb[pl.ds(q * 16, 16)] = pk >> 14
                pltpu.make_async_copy(yl_hbm.at[idxb], rb, sem).start()

        def finish(k, idxb, locb, rb, sem):
            @pl.when(k < nchunks)
            def _():
                pltpu.make_async_copy(yl_hbm.at[idxb], rb, sem).wait()

                def qgrp(q, _):
                    addr = locb[pl.ds(q * 16, 16)] * D
                    a = [addr[ln] for ln in range(16)]
                    for ln in range(16):
                        e = q * 16 + ln
                        vals = [plsc.unpack(
                                    plsc.bitcast(rb[e, pl.ds(g2 * 16, 16)],
                                                 jnp.bfloat16),
                                    format=plsc.PackFormat.INTERLEAVED)
                                for g2 in range(D // 32)]
                        for g2 in range(D // 32):
                            lo, hi = vals[g2]
                            plsc.addupdate(
                                accflat.at[pl.ds(a[ln] + g2 * 16, 16)], lo)
                            plsc.addupdate(
                                accflat.at[pl.ds(a[ln] + D // 2 + g2 * 16,
                                                 16)], hi)
                    return 0
                lax.fori_loop(0, CH // 16, qgrp, 0)

        start(jnp.int32(0), None, idxA, locA, rowsA, semA)
        start(jnp.int32(1), None, idxB, locB, rowsB, semB)

        def dpair(jj, _):
            j0 = 2 * jj
            finish(j0, idxA, locA, rowsA, semA)
            start(j0 + 2, None, idxA, locA, rowsA, semA)
            finish(j0 + 1, idxB, locB, rowsB, semB)
            start(j0 + 3, None, idxB, locB, rowsB, semB)
            return 0
        lax.fori_loop(0, (nchunks + 1) // 2, dpair, 0)

        nfc = jnp.minimum(NPT // FCH, (N - base + FCH - 1) // FCH)

        def fin(cb, _):
            n0 = base + cb * FCH
            l0 = cb * FCH
            pltpu.sync_copy(yr_hbm.at[pl.ds(n0, FCH)], ybuf)
            pltpu.sync_copy(inv_hbm.at[pl.ds(n0, FCH)], invbuf)

            def row(rr, _):
                iv = invbuf[rr, pl.ds(0, 16)]
                sums = [accflat[pl.ds((l0 + rr) * D + g * 16, 16)]
                        for g in range(D // 16)]
                for g in range(D // 16):
                    v = sums[g] * iv + ybuf[rr, pl.ds(g * 16, 16)]
                    if apply_relu:
                        v = jnp.maximum(v, 0.0)
                    obuf[rr, pl.ds(g * 16, 16)] = v
                return 0
            lax.fori_loop(0, FCH, row, 0)
            pltpu.sync_copy(obuf, out_hbm.at[pl.ds(n0, FCH)])
            return 0
        lax.fori_loop(0, nfc, fin, 0)

    return pl.kernel(
        body,
        out_type=jax.ShapeDtypeStruct((N, D), jnp.float32),
        mesh=_mesh,
        compiler_params=_params,
        scratch_types=[
            pltpu.VMEM((ACCR * D,), jnp.float32),
            pltpu.VMEM((CH, D // 2), jnp.int32),
            pltpu.VMEM((CH, D // 2), jnp.int32),
            pltpu.VMEM((CAP,), jnp.int32),
            pltpu.VMEM((CH,), jnp.int32),
            pltpu.VMEM((CH,), jnp.int32),
            pltpu.VMEM((CH,), jnp.int32),
            pltpu.VMEM((CH,), jnp.int32),
            pltpu.VMEM((16,), jnp.int32),
            pltpu.VMEM((FCH, D), jnp.float32),
            pltpu.VMEM((FCH, D), jnp.float32),
            pltpu.VMEM((FCH, 16), jnp.float32),
            pltpu.SemaphoreType.DMA,
            pltpu.SemaphoreType.DMA,
        ],
    )


_layer_relu = _make_layer(True)
_layer_last = _make_layer(False)


def kernel(x, edge_index, W1l, b1l, W1r, W2l, b2l, W2r, W3l, b3l, W3r):
    ei = edge_index.astype(jnp.int32)
    src = ei[0]
    dst = ei[1]
    elist, ecnt, inv = _bucket(src, dst)
    h = x
    layers = [
        (W1l, b1l, W1r, _layer_relu),
        (W2l, b2l, W2r, _layer_relu),
        (W3l, b3l, W3r, _layer_last),
    ]
    for Wl, bl, Wr, layer in layers:
        wt = jnp.concatenate([Wl.T, Wr.T], axis=1)
        yl, yr = _mm(h, wt, bl.reshape(1, D))
        h = layer(yl, yr, elist, ecnt, inv)
    return h
